# Initial kernel scaffold; baseline (speedup 1.0000x reference)
import functools

import jax
import jax.numpy as jnp
from jax import lax
from jax.experimental import pallas as pl
from jax.experimental.pallas import tpu as pltpu
from jax.experimental.pallas import tpu_sc as plsc

N = 320000
D = 128
G = 1024

NC = 2
NS = 16
LANES = 16
DHALF = D // NC
CHUNK = 128
NCHUNKS = N // CHUNK
GROWS = G // NS


def _body(x_hbm, idx_hbm, cnt_hbm, out_hbm, rowbuf, idxbuf, obuf, cntbuf, acc):
    c = lax.axis_index("c")
    s = lax.axis_index("s")
    col0 = c * DHALF

    def _zero_row(r, _):
        for j in range(DHALF // LANES):
            obuf[r, pl.ds(j * LANES, LANES)] = jnp.zeros((LANES,), jnp.float32)
        return 0

    lax.fori_loop(0, GROWS, _zero_row, 0)
    pltpu.sync_copy(obuf, acc.at[pl.ds(s * GROWS, GROWS)])
    plsc.subcore_barrier()

    nrem = NCHUNKS % NS
    n_i = jnp.where(s < nrem, NCHUNKS // NS + 1, NCHUNKS // NS)

    def _chunk(i, _):
        chunk_id = i * NS + s
        row0 = chunk_id * CHUNK
        pltpu.sync_copy(x_hbm.at[pl.ds(row0, CHUNK), pl.ds(col0, DHALF)], rowbuf)
        pltpu.sync_copy(idx_hbm.at[pl.ds(row0, CHUNK)], idxbuf)
        pltpu.sync_copy(rowbuf, acc.at[idxbuf], add=True)
        return 0

    lax.fori_loop(0, n_i, _chunk, 0)
    plsc.subcore_barrier()

    g0 = s * GROWS
    pltpu.sync_copy(acc.at[pl.ds(g0, GROWS)], obuf)
    pltpu.sync_copy(cnt_hbm.at[pl.ds(g0, GROWS)], cntbuf)

    def _div_row(r, _):
        cv = plsc.load_gather(cntbuf, [jnp.full((LANES,), r, jnp.int32)])
        for j in range(DHALF // LANES):
            v = obuf[r, pl.ds(j * LANES, LANES)]
            obuf[r, pl.ds(j * LANES, LANES)] = v / cv
        return 0

    lax.fori_loop(0, GROWS, _div_row, 0)
    pltpu.sync_copy(obuf, out_hbm.at[pl.ds(g0, GROWS), pl.ds(col0, DHALF)])


@jax.jit
def _pool(x, idx, cnt):
    mesh = plsc.VectorSubcoreMesh(core_axis_name="c", subcore_axis_name="s")
    return pl.kernel(
        _body,
        out_type=jax.ShapeDtypeStruct((G, D), jnp.float32),
        mesh=mesh,
        scratch_types=[
            pltpu.VMEM((CHUNK, DHALF), jnp.float32),
            pltpu.VMEM((CHUNK,), jnp.int32),
            pltpu.VMEM((GROWS, DHALF), jnp.float32),
            pltpu.VMEM((GROWS,), jnp.float32),
            pltpu.VMEM_SHARED((G, DHALF), jnp.float32),
        ],
    )(x, idx, cnt)


def kernel(input, graph_indices, node_counts):
    idx = graph_indices.astype(jnp.int32)
    return _pool(input, idx, node_counts)


# speedup vs baseline: 3.4529x; 3.4529x over previous
import functools

import jax
import jax.numpy as jnp
from jax import lax
from jax.experimental import pallas as pl
from jax.experimental.pallas import tpu as pltpu
from jax.experimental.pallas import tpu_sc as plsc

N = 320000
D = 128
G = 1024

NC = 2
NS = 16
LANES = 16
DHALF = D // NC
CHUNK = 128
NCHUNKS = N // CHUNK
GROWS = G // NS


def _body(x_hbm, idx_hbm, cnt_hbm, out_hbm, rowbuf, idxbuf, obuf, cntbuf, acc):
    c = lax.axis_index("c")
    s = lax.axis_index("s")
    col0 = c * DHALF

    def _zero_row(r, _):
        for j in range(DHALF // LANES):
            obuf[r, pl.ds(j * LANES, LANES)] = jnp.zeros((LANES,), jnp.float32)
        return 0

    lax.fori_loop(0, GROWS, _zero_row, 0)
    pltpu.sync_copy(obuf, acc.at[pl.ds(s * GROWS, GROWS)])
    plsc.subcore_barrier()

    nrem = NCHUNKS % NS
    n_i = jnp.where(s < nrem, NCHUNKS // NS + 1, NCHUNKS // NS)

    def _chunk(i, _):
        chunk_id = i * NS + s
        row0 = chunk_id * CHUNK
        pltpu.sync_copy(x_hbm.at[pl.ds(row0, CHUNK), pl.ds(col0, DHALF)], rowbuf)
        pltpu.sync_copy(idx_hbm.at[pl.ds(row0, CHUNK)], idxbuf)
        pltpu.sync_copy(rowbuf, acc.at[idxbuf], add=True)
        return 0

    lax.fori_loop(0, n_i, _chunk, 0)
    plsc.subcore_barrier()

    g0 = s * GROWS
    pltpu.sync_copy(acc.at[pl.ds(g0, GROWS)], obuf)
    pltpu.sync_copy(cnt_hbm.at[pl.ds(g0, GROWS)], cntbuf)

    def _div_row(r, _):
        cv = plsc.load_gather(cntbuf, [jnp.full((LANES,), r, jnp.int32)])
        for j in range(DHALF // LANES):
            v = obuf[r, pl.ds(j * LANES, LANES)]
            obuf[r, pl.ds(j * LANES, LANES)] = v / cv
        return 0

    lax.fori_loop(0, GROWS, _div_row, 0)
    pltpu.sync_copy(obuf, out_hbm.at[pl.ds(g0, GROWS), pl.ds(col0, DHALF)])


@jax.jit
def _pool(x, idx, cnt):
    mesh = plsc.VectorSubcoreMesh(core_axis_name="c", subcore_axis_name="s")
    return pl.kernel(
        _body,
        out_type=jax.ShapeDtypeStruct((G, D), jnp.float32),
        mesh=mesh,
        compiler_params=pltpu.CompilerParams(use_tc_tiling_on_sc=False, needs_layout_passes=False),
        scratch_types=[
            pltpu.VMEM((CHUNK, DHALF), jnp.float32),
            pltpu.VMEM((CHUNK,), jnp.int32),
            pltpu.VMEM((GROWS, DHALF), jnp.float32),
            pltpu.VMEM((GROWS,), jnp.float32),
            pltpu.VMEM_SHARED((G, DHALF), jnp.float32),
        ],
    )(x, idx, cnt)


def kernel(input, graph_indices, node_counts):
    idx = graph_indices.astype(jnp.int32)
    return _pool(input, idx, node_counts)


# speedup vs baseline: 6.2880x; 1.8211x over previous
import jax
import jax.numpy as jnp
from jax import lax
from jax.experimental import pallas as pl
from jax.experimental.pallas import tpu as pltpu
from jax.experimental.pallas import tpu_sc as plsc

N = 320000
D = 128
G = 1024

NC = 2
NS = 16
LANES = 16
DHALF = D // NC
CHUNK = 128
NCHUNKS = N // CHUNK
NBASE = NCHUNKS // NS
NREM = NCHUNKS % NS
NMAX = NBASE + 1
GROWS = G // NS


def _body(x_hbm, idx_hbm, cnt_hbm, out_hbm,
          rbuf0, rbuf1, idxbuf, obuf, cntbuf, acc,
          ld0, ld1, st0, st1):
    c = lax.axis_index("c")
    s = lax.axis_index("s")
    col0 = c * DHALF

    def _zero_row(r, _):
        for j in range(DHALF // LANES):
            obuf[r, pl.ds(j * LANES, LANES)] = jnp.zeros((LANES,), jnp.float32)
        return 0

    lax.fori_loop(0, GROWS, _zero_row, 0)
    pltpu.sync_copy(obuf, acc.at[pl.ds(s * GROWS, GROWS)])

    base = s * NBASE + jnp.minimum(s, NREM)
    n_loc = jnp.where(s < NREM, NMAX, NBASE)

    pltpu.sync_copy(idx_hbm.at[pl.ds(base, NBASE)], idxbuf.at[pl.ds(0, NBASE)])

    @pl.when(s < NREM)
    def _():
        pltpu.sync_copy(idx_hbm.at[pl.ds(base + NBASE, 1)],
                        idxbuf.at[pl.ds(NBASE, 1)])

    plsc.subcore_barrier()

    def _load(i, buf, sem):
        row0 = (base + i) * CHUNK
        pltpu.async_copy(x_hbm.at[pl.ds(row0, CHUNK), pl.ds(col0, DHALF)],
                         buf, sem)

    def _wait_load(buf, sem):
        pltpu.make_async_copy(
            x_hbm.at[pl.ds(0, CHUNK), pl.ds(col0, DHALF)], buf, sem).wait()

    _load(0, rbuf0, ld0)

    def _pair(j, _):
        a = 2 * j
        _load(a + 1, rbuf1, ld1)
        _wait_load(rbuf0, ld0)
        d0 = pltpu.async_copy(rbuf0, acc.at[idxbuf.at[a]], st0, add=True)
        d0.wait()
        _load(jnp.minimum(a + 2, n_loc - 1), rbuf0, ld0)
        _wait_load(rbuf1, ld1)
        d1 = pltpu.async_copy(rbuf1, acc.at[idxbuf.at[a + 1]], st1, add=True)
        d1.wait()
        return 0

    lax.fori_loop(0, NBASE // 2, _pair, 0)
    _wait_load(rbuf0, ld0)

    @pl.when(s < NREM)
    def _():
        pltpu.sync_copy(rbuf0, acc.at[idxbuf.at[NBASE]], add=True)

    plsc.subcore_barrier()

    g0 = s * GROWS
    pltpu.sync_copy(acc.at[pl.ds(g0, GROWS)], obuf)
    pltpu.sync_copy(cnt_hbm.at[pl.ds(g0, GROWS)], cntbuf)

    def _div_row(r, _):
        cv = plsc.load_gather(cntbuf, [jnp.full((LANES,), r, jnp.int32)])
        for j in range(DHALF // LANES):
            v = obuf[r, pl.ds(j * LANES, LANES)]
            obuf[r, pl.ds(j * LANES, LANES)] = v / cv
        return 0

    lax.fori_loop(0, GROWS, _div_row, 0)
    pltpu.sync_copy(obuf, out_hbm.at[pl.ds(g0, GROWS), pl.ds(col0, DHALF)])


@jax.jit
def _pool(x, idx, cnt):
    mesh = plsc.VectorSubcoreMesh(core_axis_name="c", subcore_axis_name="s")
    return pl.kernel(
        _body,
        out_type=jax.ShapeDtypeStruct((G, D), jnp.float32),
        mesh=mesh,
        compiler_params=pltpu.CompilerParams(
            use_tc_tiling_on_sc=False, needs_layout_passes=False),
        scratch_types=[
            pltpu.VMEM((CHUNK, DHALF), jnp.float32),
            pltpu.VMEM((CHUNK, DHALF), jnp.float32),
            pltpu.VMEM((NMAX, CHUNK), jnp.int32),
            pltpu.VMEM((GROWS, DHALF), jnp.float32),
            pltpu.VMEM((GROWS,), jnp.float32),
            pltpu.VMEM_SHARED((G, DHALF), jnp.float32),
            pltpu.SemaphoreType.DMA,
            pltpu.SemaphoreType.DMA,
            pltpu.SemaphoreType.DMA,
            pltpu.SemaphoreType.DMA,
        ],
    )(x, idx, cnt)


def kernel(input, graph_indices, node_counts):
    idx = graph_indices.astype(jnp.int32).reshape(NCHUNKS, CHUNK)
    return _pool(input, idx, node_counts)


# speedup vs baseline: 6.5208x; 1.0370x over previous
import jax
import jax.numpy as jnp
from jax import lax
from jax.experimental import pallas as pl
from jax.experimental.pallas import tpu as pltpu
from jax.experimental.pallas import tpu_sc as plsc

N = 320000
D = 128
G = 1024

NC = 2
NS = 16
LANES = 16
DHALF = D // NC
CHUNK = 128
NCHUNKS = N // CHUNK
NBASE = NCHUNKS // NS
NREM = NCHUNKS % NS
NMAX = NBASE + 1
GROWS = G // NS


def _body(x_hbm, idx_hbm, cnt_hbm, out_hbm,
          rbuf0, rbuf1, rbuf2, rbuf3, idxbuf, obuf, cntbuf, acc,
          ld0, ld1, ld2, ld3, st0, st1, st2, st3):
    c = lax.axis_index("c")
    s = lax.axis_index("s")
    col0 = c * DHALF

    def _zero_row(r, _):
        for j in range(DHALF // LANES):
            obuf[r, pl.ds(j * LANES, LANES)] = jnp.zeros((LANES,), jnp.float32)
        return 0

    lax.fori_loop(0, GROWS, _zero_row, 0)
    pltpu.sync_copy(obuf, acc.at[pl.ds(s * GROWS, GROWS)])

    base = s * NBASE + jnp.minimum(s, NREM)
    n_loc = jnp.where(s < NREM, NMAX, NBASE)

    pltpu.sync_copy(idx_hbm.at[pl.ds(base, NBASE)], idxbuf.at[pl.ds(0, NBASE)])

    @pl.when(s < NREM)
    def _():
        pltpu.sync_copy(idx_hbm.at[pl.ds(base + NBASE, 1)],
                        idxbuf.at[pl.ds(NBASE, 1)])

    plsc.subcore_barrier()

    def _load(i, buf, sem):
        row0 = (base + i) * CHUNK
        pltpu.async_copy(x_hbm.at[pl.ds(row0, CHUNK), pl.ds(col0, DHALF)],
                         buf, sem)

    def _wait_load(buf, sem):
        pltpu.make_async_copy(
            x_hbm.at[pl.ds(0, CHUNK), pl.ds(col0, DHALF)], buf, sem).wait()

    rbufs = (rbuf0, rbuf1, rbuf2, rbuf3)
    lds = (ld0, ld1, ld2, ld3)
    sts = (st0, st1, st2, st3)
    NB = 4

    for b in range(NB):
        _load(b, rbufs[b], lds[b])

    def _ring(j, _):
        a = NB * j
        descs = []
        for b in range(NB):
            _wait_load(rbufs[b], lds[b])
            descs.append(pltpu.async_copy(
                rbufs[b], acc.at[idxbuf.at[a + b]], sts[b], add=True))
        for b in range(NB):
            descs[b].wait()
            _load(jnp.minimum(a + NB + b, n_loc - 1), rbufs[b], lds[b])
        return 0

    lax.fori_loop(0, NBASE // NB, _ring, 0)
    for b in range(NB):
        _wait_load(rbufs[b], lds[b])

    @pl.when(s < NREM)
    def _():
        pltpu.sync_copy(rbuf0, acc.at[idxbuf.at[NBASE]], add=True)

    plsc.subcore_barrier()

    g0 = s * GROWS
    pltpu.sync_copy(acc.at[pl.ds(g0, GROWS)], obuf)
    pltpu.sync_copy(cnt_hbm.at[pl.ds(g0, GROWS)], cntbuf)

    def _div_row(r, _):
        cv = plsc.load_gather(cntbuf, [jnp.full((LANES,), r, jnp.int32)])
        for j in range(DHALF // LANES):
            v = obuf[r, pl.ds(j * LANES, LANES)]
            obuf[r, pl.ds(j * LANES, LANES)] = v / cv
        return 0

    lax.fori_loop(0, GROWS, _div_row, 0)
    pltpu.sync_copy(obuf, out_hbm.at[pl.ds(g0, GROWS), pl.ds(col0, DHALF)])


@jax.jit
def _pool(x, idx, cnt):
    mesh = plsc.VectorSubcoreMesh(core_axis_name="c", subcore_axis_name="s")
    return pl.kernel(
        _body,
        out_type=jax.ShapeDtypeStruct((G, D), jnp.float32),
        mesh=mesh,
        compiler_params=pltpu.CompilerParams(
            use_tc_tiling_on_sc=False, needs_layout_passes=False),
        scratch_types=[
            pltpu.VMEM((CHUNK, DHALF), jnp.float32),
            pltpu.VMEM((CHUNK, DHALF), jnp.float32),
            pltpu.VMEM((CHUNK, DHALF), jnp.float32),
            pltpu.VMEM((CHUNK, DHALF), jnp.float32),
            pltpu.VMEM((NMAX, CHUNK), jnp.int32),
            pltpu.VMEM((GROWS, DHALF), jnp.float32),
            pltpu.VMEM((GROWS,), jnp.float32),
            pltpu.VMEM_SHARED((G, DHALF), jnp.float32),
            pltpu.SemaphoreType.DMA,
            pltpu.SemaphoreType.DMA,
            pltpu.SemaphoreType.DMA,
            pltpu.SemaphoreType.DMA,
            pltpu.SemaphoreType.DMA,
            pltpu.SemaphoreType.DMA,
            pltpu.SemaphoreType.DMA,
            pltpu.SemaphoreType.DMA,
        ],
    )(x, idx, cnt)


def kernel(input, graph_indices, node_counts):
    idx = graph_indices.astype(jnp.int32).reshape(NCHUNKS, CHUNK)
    return _pool(input, idx, node_counts)
